# Initial kernel scaffold; baseline (speedup 1.0000x reference)
import functools

import jax
import jax.numpy as jnp
from jax import lax
from jax.experimental import pallas as pl
from jax.experimental.pallas import tpu as pltpu
from jax.experimental.pallas import tpu_sc as plsc

N = 10000
E = 320000
D = 128
L = 4

NC = 2
NS = 16
NW = NC * NS
EPT = E // NW
CH = 80
NCH = EPT // CH
NBUF = 5
NGRP = NCH // NBUF
RPS = N // NS


def _spmm_body(h_hbm, col_hbm, row_hbm, zeros_hbm, out_hbm,
               colv, rowv, bufs, pooled, gsems, ssems):
    cid = lax.axis_index("c")
    sid = lax.axis_index("s")
    wid = sid * NC + cid

    pltpu.sync_copy(col_hbm.at[wid], colv)
    pltpu.sync_copy(row_hbm.at[wid], rowv)

    pltpu.sync_copy(zeros_hbm.at[pl.ds(sid * RPS, RPS)],
                    pooled.at[pl.ds(sid * RPS, RPS)])
    plsc.subcore_barrier()

    def gather_start(i, b):
        pltpu.async_copy(h_hbm.at[colv.at[i]], bufs[b], gsems[b])

    def gather_wait(b):
        pltpu.make_async_copy(h_hbm.at[colv.at[0]], bufs[b], gsems[b]).wait()

    def scatter_start(i, b):
        pltpu.async_copy(bufs[b], pooled.at[rowv.at[i]], ssems[b], add=True)

    def scatter_wait(b):
        pltpu.make_async_copy(bufs[b], pooled.at[rowv.at[0]], ssems[b]).wait()

    for b in range(NBUF):
        gather_start(b, b)

    @pl.loop(0, NGRP - 1)
    def _(g):
        base = g * NBUF
        for b in range(NBUF):
            gather_wait(b)
            scatter_start(base + b, b)
        for b in range(NBUF):
            scatter_wait(b)
            gather_start(base + NBUF + b, b)

    base = (NGRP - 1) * NBUF
    for b in range(NBUF):
        gather_wait(b)
        scatter_start(base + b, b)
    for b in range(NBUF):
        scatter_wait(b)

    plsc.subcore_barrier()
    pltpu.sync_copy(pooled.at[pl.ds(sid * RPS, RPS)],
                    out_hbm.at[cid, pl.ds(sid * RPS, RPS)])


_spmm_sc = functools.partial(
    pl.kernel,
    out_type=jax.ShapeDtypeStruct((NC, N, D), jnp.float32),
    mesh=plsc.VectorSubcoreMesh(core_axis_name="c", subcore_axis_name="s"),
    scratch_types=(
        pltpu.VMEM((NCH, CH), jnp.int32),
        pltpu.VMEM((NCH, CH), jnp.int32),
        [pltpu.VMEM((CH, D), jnp.float32) for _ in range(NBUF)],
        pltpu.VMEM_SHARED((N, D), jnp.float32),
        [pltpu.SemaphoreType.DMA for _ in range(NBUF)],
        [pltpu.SemaphoreType.DMA for _ in range(NBUF)],
    ),
)(_spmm_body)


def _mlp_body(scale_ref, p0_ref, p1_ref, h_ref, w1_ref, b1_ref, w2_ref,
              b2_ref, g1_ref, be1_ref, g2_ref, be2_ref, out_ref):
    pooled = p0_ref[...] + p1_ref[...] + scale_ref[0, 0] * h_ref[...]
    t = jnp.dot(pooled, w1_ref[...], preferred_element_type=jnp.float32)
    t = t + b1_ref[...]
    mu = jnp.mean(t, axis=0, keepdims=True)
    var = jnp.mean((t - mu) ** 2, axis=0, keepdims=True)
    t = (t - mu) * lax.rsqrt(var + 1e-5) * g1_ref[...] + be1_ref[...]
    t = jnp.maximum(t, 0.0)
    t = jnp.dot(t, w2_ref[...], preferred_element_type=jnp.float32)
    t = t + b2_ref[...]
    mu = jnp.mean(t, axis=0, keepdims=True)
    var = jnp.mean((t - mu) ** 2, axis=0, keepdims=True)
    t = (t - mu) * lax.rsqrt(var + 1e-5) * g2_ref[...] + be2_ref[...]
    out_ref[...] = jnp.maximum(t, 0.0)


_mlp_tc = pl.pallas_call(
    _mlp_body,
    out_shape=jax.ShapeDtypeStruct((N, D), jnp.float32),
)


def kernel(x, edge_index, eps, W1, b1, W2, b2, bn1_g, bn1_b, bn2_g, bn2_b):
    row = edge_index[0].reshape(NW, NCH, CH)
    col = edge_index[1].reshape(NW, NCH, CH)
    zeros = jnp.zeros((N, D), jnp.float32)
    h = x
    for l in range(L):
        parts = _spmm_sc(h, col, row, zeros)
        scale = (1.0 + eps[l]).reshape(1, 1)
        h = _mlp_tc(scale, parts[0], parts[1], h,
                    W1[l], b1[l].reshape(1, D), W2[l], b2[l].reshape(1, D),
                    bn1_g[l].reshape(1, D), bn1_b[l].reshape(1, D),
                    bn2_g[l].reshape(1, D), bn2_b[l].reshape(1, D))
    return h


# speedup vs baseline: 4.8662x; 4.8662x over previous
import functools

import jax
import jax.numpy as jnp
from jax import lax
from jax.experimental import pallas as pl
from jax.experimental.pallas import tpu as pltpu
from jax.experimental.pallas import tpu_sc as plsc

N = 10000
E = 320000
D = 128
L = 4

NC = 2
NS = 16
NW = NC * NS
S = 320
NP2 = NW * S
CH = 80
GD = 4
ID = 8
NL = D // 16
MAXCH = (E + CH - 1) // CH + 1


def _seg_body(h_hbm, col_hbm, nf_hbm, lr_hbm, meta_hbm, out_hbm,
              metav, colvs, nfvs, lrvs, bufs, pooled, isems, gsems):
    cid = lax.axis_index("c")
    sid = lax.axis_index("s")
    wid = sid * NC + cid

    pltpu.sync_copy(meta_hbm.at[wid], metav)
    mv = metav[...]
    cstart = mv[0]
    nch = mv[1]
    estart = mv[2]
    eend = mv[3]

    zero16 = jnp.zeros((16,), jnp.float32)

    @pl.loop(0, S * D // 16)
    def _(i):
        pooled[pl.ds(i * 16, 16)] = zero16

    def idx_start(c, s):
        off = (cstart + c) * CH
        pltpu.async_copy(col_hbm.at[pl.ds(off, CH)], colvs[s], isems[s])
        pltpu.async_copy(nf_hbm.at[pl.ds(off, CH)], nfvs[s], isems[s])
        pltpu.async_copy(lr_hbm.at[pl.ds(off, CH)], lrvs[s], isems[s])

    def idx_wait(s):
        pltpu.make_async_copy(col_hbm.at[pl.ds(0, CH)], colvs[s], isems[s]).wait()
        pltpu.make_async_copy(nf_hbm.at[pl.ds(0, CH)], nfvs[s], isems[s]).wait()
        pltpu.make_async_copy(lr_hbm.at[pl.ds(0, CH)], lrvs[s], isems[s]).wait()

    def gather_start(b, s):
        pltpu.async_copy(h_hbm.at[colvs[s]], bufs[b], gsems[b])

    def gather_wait(b):
        pltpu.make_async_copy(h_hbm.at[colvs[0]], bufs[b], gsems[b]).wait()

    for c in range(ID):
        @pl.when(c < nch)
        def _(c=c):
            idx_start(c, c)
    for c in range(GD):
        @pl.when(c < nch)
        def _(c=c):
            idx_wait(c)
            gather_start(c, c)

    acc0 = tuple(jnp.zeros((16,), jnp.float32) for _ in range(NL))

    def chunk_step(c, b, s, acc):
        cvalid = c < nch
        ebase = (cstart + c) * CH

        @pl.when(cvalid)
        def _():
            gather_wait(b)

        def group_step(g, acc):
            iota = lax.iota(jnp.int32, 16)
            nfv16 = nfvs[s][pl.ds(g * 16, 16)]
            lrv16 = lrvs[s][pl.ds(g * 16, 16)]
            for k in range(16):
                e = g * 16 + k
                eg = ebase + e
                evalid = cvalid & (eg >= estart) & (eg < eend)
                nf = nfv16[k]
                lr = lrv16[k]
                erow = jnp.full((16,), e, jnp.int32)
                x = [plsc.load_gather(bufs[b], [erow, iota + j * 16])
                     for j in range(NL)]
                upd = tuple(acc[j] * nf + x[j] for j in range(NL))
                acc = tuple(jnp.where(evalid, upd[j], acc[j])
                            for j in range(NL))

                @pl.when(evalid & (lr >= 0))
                def _(acc=acc, lr=lr):
                    for j in range(NL):
                        pooled[pl.ds(lr * D + j * 16, 16)] = acc[j]
            return acc

        acc = pl.loop(0, CH // 16, init_carry=acc)(group_step)

        @pl.when(c + ID < nch)
        def _():
            idx_start(c + ID, s)

        @pl.when(cvalid & (c + GD < nch))
        def _():
            idx_wait((s + GD) % ID)
            gather_start(b, (s + GD) % ID)
        return acc

    def outer(gidx, acc):
        for p in range(ID):
            acc = chunk_step(gidx * ID + p, p % GD, p, acc)
        return acc

    ngrp = (nch + ID - 1) // ID
    pl.loop(0, ngrp, init_carry=acc0)(outer)

    pltpu.sync_copy(pooled, out_hbm.at[pl.ds(wid * S * D, S * D)])


_seg_sc = functools.partial(
    pl.kernel,
    out_type=jax.ShapeDtypeStruct((NP2 * D,), jnp.float32),
    mesh=plsc.VectorSubcoreMesh(core_axis_name="c", subcore_axis_name="s",
                                num_cores=NC, num_subcores=NS),
    scratch_types=(
        pltpu.VMEM((16,), jnp.int32),
        [pltpu.VMEM((CH,), jnp.int32) for _ in range(ID)],
        [pltpu.VMEM((CH,), jnp.float32) for _ in range(ID)],
        [pltpu.VMEM((CH,), jnp.int32) for _ in range(ID)],
        [pltpu.VMEM((CH, D), jnp.float32) for _ in range(GD)],
        pltpu.VMEM((S * D,), jnp.float32),
        [pltpu.SemaphoreType.DMA for _ in range(ID)],
        [pltpu.SemaphoreType.DMA for _ in range(GD)],
    ),
    compiler_params=pltpu.CompilerParams(use_tc_tiling_on_sc=False,
                                         needs_layout_passes=False),
)(_seg_body)


def _prep(edge_index):
    row = edge_index[0]
    col = edge_index[1]
    perm = jnp.argsort(row, stable=True)
    row_s = row[perm].astype(jnp.int32)
    col_s = col[perm].astype(jnp.int32)
    bounds = jnp.searchsorted(
        row_s, (jnp.arange(NW + 1) * S).astype(jnp.int32)).astype(jnp.int32)
    prev = jnp.concatenate([jnp.full((1,), -1, jnp.int32), row_s[:-1]])
    nf = (row_s == prev).astype(jnp.float32)
    nxt = jnp.concatenate([row_s[1:], jnp.full((1,), -1, jnp.int32)])
    lr = jnp.where(row_s != nxt, row_s % S, -1).astype(jnp.int32)
    pad = MAXCH * CH - E
    col_p = jnp.concatenate([col_s, jnp.zeros((pad,), jnp.int32)])
    nf_p = jnp.concatenate([nf, jnp.zeros((pad,), jnp.float32)])
    lr_p = jnp.concatenate([lr, jnp.full((pad,), -1, jnp.int32)])
    estart = bounds[:NW]
    eend = bounds[1:]
    cstart = estart // CH
    nch = jnp.where(eend > estart, (eend + CH - 1) // CH - cstart, 0)
    meta = jnp.stack([cstart, nch, estart, eend], axis=1).astype(jnp.int32)
    meta = jnp.concatenate([meta, jnp.zeros((NW, 12), jnp.int32)], axis=1)
    return col_p, nf_p, lr_p, meta


def _fc1_body(scale_ref, p_ref, h_ref, w_ref, b_ref, out_ref):
    pooled = p_ref[...] + scale_ref[0, 0] * h_ref[...]
    t = jnp.dot(pooled.astype(jnp.bfloat16), w_ref[...].astype(jnp.bfloat16),
                preferred_element_type=jnp.float32)
    out_ref[...] = t + b_ref[...]


_fc1_tc = pl.pallas_call(
    _fc1_body, out_shape=jax.ShapeDtypeStruct((N, D), jnp.float32))


def _fc2_body(t_ref, mu_ref, var_ref, g_ref, be_ref, w_ref, b_ref, out_ref):
    t = (t_ref[...] - mu_ref[...]) / jnp.sqrt(var_ref[...] + 1e-5)
    t = jnp.maximum(t * g_ref[...] + be_ref[...], 0.0)
    t = jnp.dot(t.astype(jnp.bfloat16), w_ref[...].astype(jnp.bfloat16),
                preferred_element_type=jnp.float32)
    out_ref[...] = t + b_ref[...]


_fc2_tc = pl.pallas_call(
    _fc2_body, out_shape=jax.ShapeDtypeStruct((N, D), jnp.float32))


def _norm_body(t_ref, mu_ref, var_ref, g_ref, be_ref, out_ref):
    t = (t_ref[...] - mu_ref[...]) / jnp.sqrt(var_ref[...] + 1e-5)
    out_ref[...] = jnp.maximum(t * g_ref[...] + be_ref[...], 0.0)


_norm_tc = pl.pallas_call(
    _norm_body, out_shape=jax.ShapeDtypeStruct((N, D), jnp.float32))


def kernel(x, edge_index, eps, W1, b1, W2, b2, bn1_g, bn1_b, bn2_g, bn2_b):
    col_p, nf_p, lr_p, meta = _prep(edge_index)
    h = x
    for l in range(L):
        flat = _seg_sc(h, col_p, nf_p, lr_p, meta)
        pooled = flat.reshape(NP2, D)[:N]
        scale = (1.0 + eps[l]).reshape(1, 1)
        t = _fc1_tc(scale, pooled, h, W1[l], b1[l].reshape(1, D))
        mu = jnp.mean(t, axis=0, keepdims=True)
        var = jnp.var(t, axis=0, keepdims=True)
        t = _fc2_tc(t, mu, var, bn1_g[l].reshape(1, D), bn1_b[l].reshape(1, D),
                    W2[l], b2[l].reshape(1, D))
        mu = jnp.mean(t, axis=0, keepdims=True)
        var = jnp.var(t, axis=0, keepdims=True)
        h = _norm_tc(t, mu, var, bn2_g[l].reshape(1, D),
                     bn2_b[l].reshape(1, D))
    return h
